# Initial kernel scaffold; baseline (speedup 1.0000x reference)
import jax
import jax.numpy as jnp
from jax.experimental import pallas as pl


def kernel(nodes, features, adj, local_weight):
    raise NotImplementedError("write your pallas kernel here")


# speedup vs baseline: 5.4316x; 5.4316x over previous
import jax
import jax.numpy as jnp
from jax import lax
from jax.experimental import pallas as pl
from jax.experimental.pallas import tpu as pltpu
from jax.experimental.pallas import tpu_sc as plsc

N_NODES = 10000
FEAT = 128
DEG = 32
CLIPS = 4
DIM = 64
NTOK = 8192

NC = 2
NS = 16
NW = NC * NS
CPW = FEAT // NW
L = 16

VCH = 400
NCH = N_NODES // VCH
BLK = 128
NBLK = NTOK // BLK

_INV_DEG = 1.0 / DEG

_CP = pltpu.CompilerParams(needs_layout_passes=False)


def _neigh_body(adjT_hbm, nodes_hbm, neighT_hbm, row_s, nodes_s, blk_s, sem):
    j = lax.axis_index("s") * NC + lax.axis_index("c")
    pltpu.sync_copy(adjT_hbm.at[j], row_s)
    pltpu.sync_copy(nodes_hbm, nodes_s)

    def blk_body(b, _):
        def i_body(ic, _):
            nv = nodes_s[pl.ds(b * BLK + ic * L, L)]
            blk_s[pl.ds(ic * L, L)] = plsc.load_gather(row_s, [nv])
            return _

        lax.fori_loop(0, BLK // L, i_body, None)
        pltpu.sync_copy(blk_s, neighT_hbm.at[b, j])
        return _

    lax.fori_loop(0, NBLK, blk_body, None)


def _sc_body(featT_hbm, adjc_hbm, neighT_hbm, out_hbm,
             feat_s, h1_s, adjc_s, neighc_s, featb_s, sem):
    wid = lax.axis_index("s") * NC + lax.axis_index("c")
    lane = lax.iota(jnp.int32, L)
    cvecs = [jnp.full((L,), c, jnp.int32) for c in range(CPW)]

    pltpu.sync_copy(featT_hbm.at[wid], feat_s)

    def chunk_body(ch, _):
        pltpu.sync_copy(adjc_hbm.at[ch], adjc_s)

        def v_body(vc, _):
            base = vc * L
            acc = [jnp.zeros((L,), jnp.float32) for _ in range(CPW)]
            for j in range(DEG):
                nidx = adjc_s[j, pl.ds(base, L)]
                for c in range(CPW):
                    acc[c] = acc[c] + plsc.load_gather(feat_s, [cvecs[c], nidx])
            row = ch * VCH + base
            for c in range(CPW):
                h1_s[c, pl.ds(row, L)] = acc[c] * _INV_DEG
            return _

        lax.fori_loop(0, VCH // L, v_body, None)
        return _

    lax.fori_loop(0, NCH, chunk_body, None)

    def blk_body(b, _):
        pltpu.sync_copy(neighT_hbm.at[b], neighc_s)

        def i_body(ic, _):
            acc = [jnp.zeros((L,), jnp.float32) for _ in range(CPW)]
            for j in range(DEG):
                nidx = neighc_s[j, pl.ds(ic * L, L)]
                for c in range(CPW):
                    acc[c] = acc[c] + plsc.load_gather(h1_s, [cvecs[c], nidx])
            for c in range(CPW):
                featb_s[c, pl.ds(ic * L, L)] = acc[c] * _INV_DEG
            return _

        lax.fori_loop(0, BLK // L, i_body, None)
        pltpu.sync_copy(featb_s, out_hbm.at[wid, b])
        return _

    lax.fori_loop(0, NBLK, blk_body, None)


def _sc_aggregate(featT, adjT, adjc, nodes_flat):
    mesh = plsc.VectorSubcoreMesh(core_axis_name="c", subcore_axis_name="s")
    neigh_fn = pl.kernel(
        _neigh_body,
        out_type=jax.ShapeDtypeStruct((NBLK, DEG, BLK), jnp.int32),
        mesh=mesh,
        compiler_params=_CP,
        scratch_types=[
            pltpu.VMEM((N_NODES,), jnp.int32),
            pltpu.VMEM((NTOK,), jnp.int32),
            pltpu.VMEM((BLK,), jnp.int32),
            pltpu.SemaphoreType.DMA,
        ],
    )
    neighT = neigh_fn(adjT, nodes_flat)

    main_fn = pl.kernel(
        _sc_body,
        out_type=jax.ShapeDtypeStruct((NW, NBLK, CPW, BLK), jnp.float32),
        mesh=mesh,
        compiler_params=_CP,
        scratch_types=[
            pltpu.VMEM((CPW, N_NODES), jnp.float32),
            pltpu.VMEM((CPW, N_NODES), jnp.float32),
            pltpu.VMEM((DEG, VCH), jnp.int32),
            pltpu.VMEM((DEG, BLK), jnp.int32),
            pltpu.VMEM((CPW, BLK), jnp.float32),
            pltpu.SemaphoreType.DMA,
        ],
    )
    return main_fn(featT, adjc, neighT)


def _mm_body(x_ref, w_ref, o_ref):
    o_ref[...] = jnp.maximum(
        jnp.dot(x_ref[...], w_ref[...], preferred_element_type=jnp.float32), 0.0)


def _project(feat, wt):
    return pl.pallas_call(
        _mm_body,
        grid=(8,),
        in_specs=[
            pl.BlockSpec((NTOK // 8, FEAT), lambda i: (i, 0)),
            pl.BlockSpec((FEAT, CLIPS * DIM), lambda i: (0, 0)),
        ],
        out_specs=pl.BlockSpec((NTOK // 8, CLIPS * DIM), lambda i: (i, 0)),
        out_shape=jax.ShapeDtypeStruct((NTOK, CLIPS * DIM), jnp.float32),
    )(feat, wt)


def kernel(nodes, features, adj, local_weight):
    nodes_flat = nodes.reshape(-1).astype(jnp.int32)
    adj = adj.astype(jnp.int32)
    featT = features.T.reshape(NW, CPW, N_NODES)
    adjT = adj.T
    adjc = adjT.reshape(DEG, NCH, VCH).transpose(1, 0, 2)

    featT_out = _sc_aggregate(featT, adjT, adjc, nodes_flat)
    feat = featT_out.transpose(1, 3, 0, 2).reshape(NTOK, FEAT)

    wt = local_weight.reshape(CLIPS * DIM, FEAT).T
    out = _project(feat, wt)
    return out.reshape(nodes.shape[0], nodes.shape[1], CLIPS, DIM)
